# baseline (speedup 1.0000x reference)
import functools

import jax
import jax.numpy as jnp
import numpy as np
from jax import lax
from jax.experimental import pallas as pl
from jax.experimental.pallas import tpu as pltpu
from jax.experimental.pallas import tpu_sc as plsc

_B = 16384
_F = 26
_D = 16
_E = _F * _D
_NW = 32
_BPW = _B // _NW

_MS = (np.arange(_E)[None, :] % _D == np.arange(_D)[:, None]).astype(np.float32)


def _fo_body(idx_hbm, w_hbm, out_hbm, idx_v, vals_v, acc_v, sem):
    wid = lax.axis_index("s") * 2 + lax.axis_index("c")
    pltpu.sync_copy(idx_hbm.at[wid], idx_v)
    pltpu.async_copy(w_hbm.at[idx_v], vals_v, sem).wait()
    for c in range(_BPW // 16):
        v = vals_v[pl.ds(c * 16, 16)]
        for f in range(1, _F):
            v = v + vals_v[pl.ds(f * _BPW + c * 16, 16)]
        acc_v[pl.ds(c * 16, 16)] = v
    pltpu.sync_copy(acc_v, out_hbm.at[pl.ds(wid * _BPW, _BPW)])


def _first_order(idx, w):
    fo_kernel = functools.partial(
        pl.kernel,
        out_type=jax.ShapeDtypeStruct((_B,), jnp.float32),
        mesh=plsc.VectorSubcoreMesh(core_axis_name="c", subcore_axis_name="s"),
        scratch_types=[
            pltpu.VMEM((_F * _BPW,), jnp.int32),
            pltpu.VMEM((_F * _BPW,), jnp.float32),
            pltpu.VMEM((_BPW,), jnp.float32),
            pltpu.SemaphoreType.DMA,
        ],
    )(_fo_body)
    return fo_kernel(idx, w)


def _so_body(xt_ref, ms_ref, o_ref):
    bc = o_ref.shape[0]
    cw = 512
    gc = 104
    for c in range(bc // cw):
        cs = pl.ds(c * cw, cw)
        q = jnp.zeros((cw,), jnp.float32)
        s = jnp.zeros((_D, cw), jnp.float32)
        for g in range(_E // gc):
            xg = xt_ref[pl.ds(g * gc, gc), cs]
            q = q + jnp.sum(xg * xg, axis=0)
            s = s + jnp.dot(ms_ref[:, pl.ds(g * gc, gc)], xg,
                            preferred_element_type=jnp.float32)
        ssq = jnp.sum(s * s, axis=0)
        o_ref[cs] = 0.5 * (ssq - q)


def _comb_body(a_ref, b_ref, o_ref):
    o_ref[...] = a_ref[...] + b_ref[...]


def _wflat_body(w_ref, o_ref, sem):
    cp = pltpu.make_async_copy(w_ref.at[:, 0], o_ref, sem)
    cp.start()
    cp.wait()


def _wflat(w):
    return pl.pallas_call(
        _wflat_body,
        in_specs=[pl.BlockSpec(memory_space=pltpu.MemorySpace.HBM)],
        out_specs=pl.BlockSpec(memory_space=pltpu.MemorySpace.HBM),
        out_shape=jax.ShapeDtypeStruct((w.shape[0],), jnp.float32),
        scratch_shapes=[pltpu.SemaphoreType.DMA],
    )(w)


def kernel(sparse_inputs, embed_inputs, w):
    idx = sparse_inputs.T.reshape(_F, _NW, _BPW).transpose(1, 0, 2).reshape(_NW, _F * _BPW)
    fo = _first_order(idx, w[:, 0])

    xt = embed_inputs.reshape(_B, _E).T
    blk = 2048
    so = pl.pallas_call(
        _so_body,
        grid=(_B // blk,),
        in_specs=[
            pl.BlockSpec((_E, blk), lambda i: (0, i)),
            pl.BlockSpec((_D, _E), lambda i: (0, 0)),
        ],
        out_specs=pl.BlockSpec((blk,), lambda i: (i,)),
        out_shape=jax.ShapeDtypeStruct((_B,), jnp.float32),
    )(xt, jnp.asarray(_MS))

    out = pl.pallas_call(
        _comb_body,
        in_specs=[
            pl.BlockSpec((_B,), lambda: (0,)),
            pl.BlockSpec((_B,), lambda: (0,)),
        ],
        out_specs=pl.BlockSpec((_B,), lambda: (0,)),
        out_shape=jax.ShapeDtypeStruct((_B,), jnp.float32),
    )(fo, so)
    return out.reshape(_B, 1)


# speedup vs baseline: 1.5765x; 1.5765x over previous
import functools

import jax
import jax.numpy as jnp
import numpy as np
from jax import lax
from jax.experimental import pallas as pl
from jax.experimental.pallas import tpu as pltpu
from jax.experimental.pallas import tpu_sc as plsc

_B = 16384
_F = 26
_D = 16
_E = _F * _D
_NW = 32
_BPW = _B // _NW

_MS = (np.arange(_E)[None, :] % _D == np.arange(_D)[:, None]).astype(np.float32)


_WPAD = 1 << 20


def _fo_body(idx_hbm, w_hbm, out_hbm, idx_v, vals_v, acc_v, sem):
    wid = lax.axis_index("s") * 2 + lax.axis_index("c")
    pltpu.sync_copy(idx_hbm.at[wid], idx_v)
    pltpu.async_copy(w_hbm.at[idx_v], vals_v, sem).wait()
    for c in range(_BPW // 16):
        v = vals_v[pl.ds(c * 16, 16)]
        for f in range(1, _F):
            v = v + vals_v[pl.ds(f * _BPW + c * 16, 16)]
        acc_v[pl.ds(c * 16, 16)] = v
    pltpu.sync_copy(acc_v, out_hbm.at[pl.ds(wid * _BPW, _BPW)])


def _first_order(idx, w):
    fo_kernel = functools.partial(
        pl.kernel,
        out_type=jax.ShapeDtypeStruct((_B,), jnp.float32),
        mesh=plsc.VectorSubcoreMesh(core_axis_name="c", subcore_axis_name="s"),
        scratch_types=[
            pltpu.VMEM((_F * _BPW,), jnp.int32),
            pltpu.VMEM((_F * _BPW,), jnp.float32),
            pltpu.VMEM((_BPW,), jnp.float32),
            pltpu.SemaphoreType.DMA,
        ],
    )(_fo_body)
    return fo_kernel(idx, w)


def _so_body(xt_ref, ms_ref, o_ref):
    bc = o_ref.shape[0]
    cw = 512
    gc = 104
    for c in range(bc // cw):
        cs = pl.ds(c * cw, cw)
        q = jnp.zeros((cw,), jnp.float32)
        s = jnp.zeros((_D, cw), jnp.float32)
        for g in range(_E // gc):
            xg = xt_ref[pl.ds(g * gc, gc), cs]
            q = q + jnp.sum(xg * xg, axis=0)
            s = s + jnp.dot(ms_ref[:, pl.ds(g * gc, gc)], xg,
                            preferred_element_type=jnp.float32)
        ssq = jnp.sum(s * s, axis=0)
        o_ref[cs] = 0.5 * (ssq - q)


def _comb_body(a_ref, b_ref, o_ref):
    o_ref[...] = a_ref[...] + b_ref[...]


def _wflat_body(w_ref, o_ref, sem):
    cp = pltpu.make_async_copy(w_ref.at[:, 0], o_ref, sem)
    cp.start()
    cp.wait()


def _wflat(w):
    return pl.pallas_call(
        _wflat_body,
        in_specs=[pl.BlockSpec(memory_space=pltpu.MemorySpace.HBM)],
        out_specs=pl.BlockSpec(memory_space=pltpu.MemorySpace.HBM),
        out_shape=jax.ShapeDtypeStruct((w.shape[0],), jnp.float32),
        scratch_shapes=[pltpu.SemaphoreType.DMA],
    )(w)


def kernel(sparse_inputs, embed_inputs, w):
    idx = sparse_inputs.T.reshape(_F, _NW, _BPW).transpose(1, 0, 2).reshape(_NW, _F * _BPW)
    wp = jnp.concatenate(
        [w, jnp.zeros((_WPAD - w.shape[0], 1), jnp.float32)], axis=0).reshape(-1)
    fo = _first_order(idx, wp)

    xt = embed_inputs.reshape(_B, _E).T
    blk = 2048
    so = pl.pallas_call(
        _so_body,
        grid=(_B // blk,),
        in_specs=[
            pl.BlockSpec((_E, blk), lambda i: (0, i)),
            pl.BlockSpec((_D, _E), lambda i: (0, 0)),
        ],
        out_specs=pl.BlockSpec((blk,), lambda i: (i,)),
        out_shape=jax.ShapeDtypeStruct((_B,), jnp.float32),
    )(xt, jnp.asarray(_MS))

    out = pl.pallas_call(
        _comb_body,
        in_specs=[
            pl.BlockSpec((_B,), lambda: (0,)),
            pl.BlockSpec((_B,), lambda: (0,)),
        ],
        out_specs=pl.BlockSpec((_B,), lambda: (0,)),
        out_shape=jax.ShapeDtypeStruct((_B,), jnp.float32),
    )(fo, so)
    return out.reshape(_B, 1)
